# baseline (speedup 1.0000x reference)
import functools

import jax
import jax.numpy as jnp
from jax import lax
from jax.experimental import pallas as pl
from jax.experimental.pallas import tpu as pltpu
from jax.experimental.pallas import tpu_sc as plsc

_B, _N, _D = 16, 196, 768
_ND = 64
_P, _LEN, _TOPK = 32768, 2, 8
_LG, _H = 20, 8
_HD = _D // _H
_ROWS = _B * _TOPK
_NW = 16
_RPW = _ROWS // _NW
_RD = _LEN * _D


def _sc_gather_body(prompt_hbm, pkey_hbm, idx_hbm, rows_out, keys_out,
                    idx_v, rows_v, keys_v, sem):
    wid = lax.axis_index("s") * 2 + lax.axis_index("c")

    @pl.when(wid < _NW)
    def _():
        base = wid * _RPW
        pltpu.sync_copy(idx_hbm.at[pl.ds(base, _RPW)], idx_v)
        pltpu.async_copy(prompt_hbm.at[idx_v], rows_v, sem).wait()
        pltpu.async_copy(pkey_hbm.at[idx_v], keys_v, sem).wait()
        pltpu.sync_copy(rows_v, rows_out.at[pl.ds(base, _RPW)])
        pltpu.sync_copy(keys_v, keys_out.at[pl.ds(base, _RPW)])


@functools.cache
def _sc_gather():
    return pl.kernel(
        _sc_gather_body,
        out_type=(
            jax.ShapeDtypeStruct((_ROWS, _RD), jnp.float32),
            jax.ShapeDtypeStruct((_ROWS, _D), jnp.float32),
        ),
        mesh=plsc.VectorSubcoreMesh(core_axis_name="c", subcore_axis_name="s"),
        scratch_types=[
            pltpu.VMEM((_RPW,), jnp.int32),
            pltpu.VMEM((_RPW, _RD), jnp.float32),
            pltpu.VMEM((_RPW, _D), jnp.float32),
            pltpu.SemaphoreType.DMA,
        ],
    )


def _tc_body(x_ref, depth_ref, rows_ref, keys_ref, gp_ref,
             wq_ref, bq_ref, wkv_ref, bkv_ref, wproj_ref, bproj_ref,
             out_ref, bkn_ref, sim_ref, q_scratch):
    b = pl.program_id(0)

    x = x_ref[0]
    xm = jnp.mean(x, axis=0, keepdims=True)
    xn = xm * lax.rsqrt(jnp.maximum(jnp.sum(xm * xm), 1e-12))

    keys = keys_ref[0]
    ksq = jnp.sum(keys * keys, axis=1, keepdims=True)
    kn = keys * lax.rsqrt(jnp.maximum(ksq, 1e-12))
    bkn_ref[0] = kn

    @pl.when(b == 0)
    def _():
        sim_ref[0, 0] = 0.0
        q_scratch[...] = lax.dot_general(
            gp_ref[0], wq_ref[...], (((1,), (1,)), ((), ())),
            preferred_element_type=jnp.float32) + bq_ref[...]

    sim_ref[0, 0] += jnp.sum(kn * xn) * (1.0 / _B)

    kv = lax.dot_general(
        depth_ref[0], wkv_ref[...], (((1,), (1,)), ((), ())),
        preferred_element_type=jnp.float32) + bkv_ref[...]

    scale = float(_HD) ** -0.5
    outs = []
    for h in range(_H):
        qh = q_scratch[:, h * _HD:(h + 1) * _HD]
        kh = kv[:, h * _HD:(h + 1) * _HD]
        vh = kv[:, _D + h * _HD:_D + (h + 1) * _HD]
        s = lax.dot_general(qh, kh, (((1,), (1,)), ((), ())),
                            preferred_element_type=jnp.float32) * scale
        s = s - jnp.max(s, axis=1, keepdims=True)
        e = jnp.exp(s)
        p = e * (1.0 / jnp.sum(e, axis=1, keepdims=True))
        outs.append(lax.dot_general(p, vh, (((1,), (0,)), ((), ())),
                                    preferred_element_type=jnp.float32))
    o = jnp.concatenate(outs, axis=1)
    ca = lax.dot_general(o, wproj_ref[...], (((1,), (1,)), ((), ())),
                         preferred_element_type=jnp.float32) + bproj_ref[...]

    out_ref[0, 0:_LG] = ca
    out_ref[0, _LG:_LG + _TOPK * _LEN] = rows_ref[0]
    out_ref[0, _LG + _TOPK * _LEN:] = x


def _dense_tc(x_embed, depth_feature, rows, keys, g_prompt,
              Wq, bq, Wkv, bkv, Wproj, bproj, interpret=False):
    n_out = _LG + _TOPK * _LEN + _N
    grid = (_B,)
    return pl.pallas_call(
        _tc_body,
        grid=grid,
        in_specs=[
            pl.BlockSpec((1, _N, _D), lambda b: (b, 0, 0)),
            pl.BlockSpec((1, _ND, _D), lambda b: (b, 0, 0)),
            pl.BlockSpec((1, _TOPK * _LEN, _D), lambda b: (b, 0, 0)),
            pl.BlockSpec((1, _TOPK, _D), lambda b: (b, 0, 0)),
            pl.BlockSpec((1, _LG, _D), lambda b: (0, 0, 0)),
            pl.BlockSpec((_D, _D), lambda b: (0, 0)),
            pl.BlockSpec((1, _D), lambda b: (0, 0)),
            pl.BlockSpec((2 * _D, _D), lambda b: (0, 0)),
            pl.BlockSpec((1, 2 * _D), lambda b: (0, 0)),
            pl.BlockSpec((_D, _D), lambda b: (0, 0)),
            pl.BlockSpec((1, _D), lambda b: (0, 0)),
        ],
        out_specs=[
            pl.BlockSpec((1, n_out, _D), lambda b: (b, 0, 0)),
            pl.BlockSpec((1, _TOPK, _D), lambda b: (b, 0, 0)),
            pl.BlockSpec((1, 1), lambda b: (0, 0), memory_space=pltpu.SMEM),
        ],
        out_shape=[
            jax.ShapeDtypeStruct((_B, n_out, _D), jnp.float32),
            jax.ShapeDtypeStruct((_B, _TOPK, _D), jnp.float32),
            jax.ShapeDtypeStruct((1, 1), jnp.float32),
        ],
        scratch_shapes=[pltpu.VMEM((_LG, _D), jnp.float32)],
        interpret=interpret,
    )(x_embed, depth_feature, rows, keys, g_prompt,
      Wq, bq.reshape(1, _D), Wkv, bkv.reshape(1, 2 * _D),
      Wproj, bproj.reshape(1, _D))


def kernel(x_embed, prompt_mask, depth_feature, prompt, prompt_key,
           prompt_key_g, g_prompt, Wq, bq, Wkv, bkv, Wproj, bproj):
    idx = prompt_mask.reshape(_ROWS)
    rows, keys = _sc_gather()(prompt.reshape(_P, _RD), prompt_key, idx)
    rows = rows.reshape(_B, _TOPK * _LEN, _D)
    keys = keys.reshape(_B, _TOPK, _D)
    prompted, bkn, sim = _dense_tc(
        x_embed, depth_feature, rows, keys, g_prompt,
        Wq, bq, Wkv, bkv, Wproj, bproj)
    return prompted, bkn, sim.reshape(())


# speedup vs baseline: 2.4511x; 2.4511x over previous
import functools

import jax
import jax.numpy as jnp
from jax import lax
from jax.experimental import pallas as pl
from jax.experimental.pallas import tpu as pltpu
from jax.experimental.pallas import tpu_sc as plsc

_B, _N, _D = 16, 196, 768
_ND = 64
_P, _LEN, _TOPK = 32768, 2, 8
_LG, _H = 20, 8
_HD = _D // _H
_ROWS = _B * _TOPK
_NW = 16
_RPW = _ROWS // _NW
_RD = _LEN * _D


def _sc_gather_body(prompt_hbm, pkey_hbm, idx_hbm, rows_out, keys_out,
                    idx_v, rows_v, keys_v, sem):
    wid = lax.axis_index("s") * 2 + lax.axis_index("c")

    @pl.when(wid < _NW)
    def _():
        base = wid * _RPW
        pltpu.sync_copy(idx_hbm.at[pl.ds(base, _RPW)], idx_v)
        pltpu.async_copy(prompt_hbm.at[idx_v], rows_v, sem).wait()
        pltpu.async_copy(pkey_hbm.at[idx_v], keys_v, sem).wait()
        pltpu.sync_copy(rows_v, rows_out.at[pl.ds(base, _RPW)])
        pltpu.sync_copy(keys_v, keys_out.at[pl.ds(base, _RPW)])


@functools.cache
def _sc_gather():
    return pl.kernel(
        _sc_gather_body,
        out_type=(
            jax.ShapeDtypeStruct((_ROWS, _LEN, _D), jnp.float32),
            jax.ShapeDtypeStruct((_ROWS, _D), jnp.float32),
        ),
        mesh=plsc.VectorSubcoreMesh(core_axis_name="c", subcore_axis_name="s"),
        scratch_types=[
            pltpu.VMEM((_RPW,), jnp.int32),
            pltpu.VMEM((_RPW, _LEN, _D), jnp.float32),
            pltpu.VMEM((_RPW, _D), jnp.float32),
            pltpu.SemaphoreType.DMA,
        ],
        compiler_params=pltpu.CompilerParams(use_tc_tiling_on_sc=True),
    )


def _tc_body(x_ref, depth_ref, rows_ref, keys_ref, gp_ref,
             wq_ref, bq_ref, wkv_ref, bkv_ref, wproj_ref, bproj_ref,
             out_ref, bkn_ref, sim_ref, q_scratch):
    b = pl.program_id(0)

    x = x_ref[0]
    xm = jnp.mean(x, axis=0, keepdims=True)
    xn = xm * lax.rsqrt(jnp.maximum(jnp.sum(xm * xm), 1e-12))

    keys = keys_ref[0]
    ksq = jnp.sum(keys * keys, axis=1, keepdims=True)
    kn = keys * lax.rsqrt(jnp.maximum(ksq, 1e-12))
    bkn_ref[0] = kn

    @pl.when(b == 0)
    def _():
        sim_ref[0, 0] = 0.0
        q_scratch[...] = lax.dot_general(
            gp_ref[0], wq_ref[...], (((1,), (1,)), ((), ())),
            preferred_element_type=jnp.float32) + bq_ref[...]

    sim_ref[0, 0] += jnp.sum(kn * xn) * (1.0 / _B)

    kv = lax.dot_general(
        depth_ref[0], wkv_ref[...], (((1,), (1,)), ((), ())),
        preferred_element_type=jnp.float32) + bkv_ref[...]

    scale = float(_HD) ** -0.5
    outs = []
    for h in range(_H):
        qh = q_scratch[:, h * _HD:(h + 1) * _HD]
        kh = kv[:, h * _HD:(h + 1) * _HD]
        vh = kv[:, _D + h * _HD:_D + (h + 1) * _HD]
        s = lax.dot_general(qh, kh, (((1,), (1,)), ((), ())),
                            preferred_element_type=jnp.float32) * scale
        s = s - jnp.max(s, axis=1, keepdims=True)
        e = jnp.exp(s)
        p = e * (1.0 / jnp.sum(e, axis=1, keepdims=True))
        outs.append(lax.dot_general(p, vh, (((1,), (0,)), ((), ())),
                                    preferred_element_type=jnp.float32))
    o = jnp.concatenate(outs, axis=1)
    ca = lax.dot_general(o, wproj_ref[...], (((1,), (1,)), ((), ())),
                         preferred_element_type=jnp.float32) + bproj_ref[...]

    out_ref[0, 0:_LG] = ca
    out_ref[0, _LG:_LG + _TOPK * _LEN] = rows_ref[0]
    out_ref[0, _LG + _TOPK * _LEN:] = x


def _dense_tc(x_embed, depth_feature, rows, keys, g_prompt,
              Wq, bq, Wkv, bkv, Wproj, bproj, interpret=False):
    n_out = _LG + _TOPK * _LEN + _N
    grid = (_B,)
    return pl.pallas_call(
        _tc_body,
        grid=grid,
        in_specs=[
            pl.BlockSpec((1, _N, _D), lambda b: (b, 0, 0)),
            pl.BlockSpec((1, _ND, _D), lambda b: (b, 0, 0)),
            pl.BlockSpec((1, _TOPK * _LEN, _D), lambda b: (b, 0, 0)),
            pl.BlockSpec((1, _TOPK, _D), lambda b: (b, 0, 0)),
            pl.BlockSpec((1, _LG, _D), lambda b: (0, 0, 0)),
            pl.BlockSpec((_D, _D), lambda b: (0, 0)),
            pl.BlockSpec((1, _D), lambda b: (0, 0)),
            pl.BlockSpec((2 * _D, _D), lambda b: (0, 0)),
            pl.BlockSpec((1, 2 * _D), lambda b: (0, 0)),
            pl.BlockSpec((_D, _D), lambda b: (0, 0)),
            pl.BlockSpec((1, _D), lambda b: (0, 0)),
        ],
        out_specs=[
            pl.BlockSpec((1, n_out, _D), lambda b: (b, 0, 0)),
            pl.BlockSpec((1, _TOPK, _D), lambda b: (b, 0, 0)),
            pl.BlockSpec((1, 1), lambda b: (0, 0), memory_space=pltpu.SMEM),
        ],
        out_shape=[
            jax.ShapeDtypeStruct((_B, n_out, _D), jnp.float32),
            jax.ShapeDtypeStruct((_B, _TOPK, _D), jnp.float32),
            jax.ShapeDtypeStruct((1, 1), jnp.float32),
        ],
        scratch_shapes=[pltpu.VMEM((_LG, _D), jnp.float32)],
        interpret=interpret,
    )(x_embed, depth_feature, rows, keys, g_prompt,
      Wq, bq.reshape(1, _D), Wkv, bkv.reshape(1, 2 * _D),
      Wproj, bproj.reshape(1, _D))


def kernel(x_embed, prompt_mask, depth_feature, prompt, prompt_key,
           prompt_key_g, g_prompt, Wq, bq, Wkv, bkv, Wproj, bproj):
    idx = prompt_mask.reshape(_ROWS)
    rows, keys = _sc_gather()(prompt, prompt_key, idx)
    rows = rows.reshape(_B, _TOPK * _LEN, _D)
    keys = keys.reshape(_B, _TOPK, _D)
    prompted, bkn, sim = _dense_tc(
        x_embed, depth_feature, rows, keys, g_prompt,
        Wq, bq, Wkv, bkv, Wproj, bproj)
    return prompted, bkn, sim.reshape(())


# speedup vs baseline: 2.5798x; 1.0525x over previous
import functools

import jax
import jax.numpy as jnp
from jax import lax
from jax.experimental import pallas as pl
from jax.experimental.pallas import tpu as pltpu
from jax.experimental.pallas import tpu_sc as plsc

_B, _N, _D = 16, 196, 768
_ND = 64
_P, _LEN, _TOPK = 32768, 2, 8
_LG, _H = 20, 8
_HD = _D // _H
_ROWS = _B * _TOPK
_NW = 16
_RPW = _ROWS // _NW
_BPS = 2


def _sc_gather_body(prompt_hbm, pkey_hbm, idx_hbm, rows_out, keys_out,
                    idx_v, rows_v, keys_v, sem):
    wid = lax.axis_index("s") * 2 + lax.axis_index("c")

    @pl.when(wid < _NW)
    def _():
        base = wid * _RPW
        pltpu.sync_copy(idx_hbm.at[pl.ds(base, _RPW)], idx_v)
        pltpu.async_copy(prompt_hbm.at[idx_v], rows_v, sem).wait()
        pltpu.async_copy(pkey_hbm.at[idx_v], keys_v, sem).wait()
        pltpu.sync_copy(rows_v, rows_out.at[pl.ds(base, _RPW)])
        pltpu.sync_copy(keys_v, keys_out.at[pl.ds(base, _RPW)])


@functools.cache
def _sc_gather():
    return pl.kernel(
        _sc_gather_body,
        out_type=(
            jax.ShapeDtypeStruct((_ROWS, _LEN, _D), jnp.float32),
            jax.ShapeDtypeStruct((_ROWS, _D), jnp.float32),
        ),
        mesh=plsc.VectorSubcoreMesh(core_axis_name="c", subcore_axis_name="s"),
        scratch_types=[
            pltpu.VMEM((_RPW,), jnp.int32),
            pltpu.VMEM((_RPW, _LEN, _D), jnp.float32),
            pltpu.VMEM((_RPW, _D), jnp.float32),
            pltpu.SemaphoreType.DMA,
        ],
        compiler_params=pltpu.CompilerParams(use_tc_tiling_on_sc=True),
    )


def _tc_body(x_ref, depth_ref, rows_ref, keys_ref, gp_ref,
             wq_ref, bq_ref, wkv_ref, bkv_ref, wproj_ref, bproj_ref,
             out_ref, bkn_ref, sim_ref, q_scratch, wkv_b, wproj_b):
    g = pl.program_id(0)

    @pl.when(g == 0)
    def _():
        sim_ref[0, 0] = 0.0
        q = lax.dot_general(
            gp_ref[0], wq_ref[...], (((1,), (1,)), ((), ())),
            preferred_element_type=jnp.float32) + bq_ref[...]
        q_scratch[...] = q.astype(jnp.bfloat16)
        wkv_b[...] = wkv_ref[...].astype(jnp.bfloat16)
        wproj_b[...] = wproj_ref[...].astype(jnp.bfloat16)

    scale = float(_HD) ** -0.5
    sim_acc = jnp.float32(0.0)
    for i in range(_BPS):
        x = x_ref[i]
        xm = jnp.mean(x, axis=0, keepdims=True)
        xn = xm * lax.rsqrt(jnp.maximum(jnp.sum(xm * xm), 1e-12))

        keys = keys_ref[i]
        ksq = jnp.sum(keys * keys, axis=1, keepdims=True)
        kn = keys * lax.rsqrt(jnp.maximum(ksq, 1e-12))
        bkn_ref[i] = kn
        sim_acc += jnp.sum(kn * xn)

        kv = lax.dot_general(
            depth_ref[i].astype(jnp.bfloat16), wkv_b[...],
            (((1,), (1,)), ((), ())),
            preferred_element_type=jnp.float32) + bkv_ref[...]
        kvb = kv.astype(jnp.bfloat16)

        outs = []
        for h in range(_H):
            qh = q_scratch[:, h * _HD:(h + 1) * _HD]
            kh = kvb[:, h * _HD:(h + 1) * _HD]
            vh = kvb[:, _D + h * _HD:_D + (h + 1) * _HD]
            s = lax.dot_general(qh, kh, (((1,), (1,)), ((), ())),
                                preferred_element_type=jnp.float32) * scale
            s = s - jnp.max(s, axis=1, keepdims=True)
            e = jnp.exp(s)
            p = (e * (1.0 / jnp.sum(e, axis=1, keepdims=True))
                 ).astype(jnp.bfloat16)
            outs.append(lax.dot_general(p, vh, (((1,), (0,)), ((), ())),
                                        preferred_element_type=jnp.float32))
        o = jnp.concatenate(outs, axis=1).astype(jnp.bfloat16)
        ca = lax.dot_general(o, wproj_b[...], (((1,), (1,)), ((), ())),
                             preferred_element_type=jnp.float32) + bproj_ref[...]

        out_ref[i, 0:_LG] = ca
        out_ref[i, _LG:_LG + _TOPK * _LEN] = rows_ref[i]
        out_ref[i, _LG + _TOPK * _LEN:] = x
    sim_ref[0, 0] += sim_acc * (1.0 / _B)


def _dense_tc(x_embed, depth_feature, rows, keys, g_prompt,
              Wq, bq, Wkv, bkv, Wproj, bproj, interpret=False):
    n_out = _LG + _TOPK * _LEN + _N
    grid = (_B // _BPS,)
    return pl.pallas_call(
        _tc_body,
        grid=grid,
        in_specs=[
            pl.BlockSpec((_BPS, _N, _D), lambda b: (b, 0, 0)),
            pl.BlockSpec((_BPS, _ND, _D), lambda b: (b, 0, 0)),
            pl.BlockSpec((_BPS, _TOPK * _LEN, _D), lambda b: (b, 0, 0)),
            pl.BlockSpec((_BPS, _TOPK, _D), lambda b: (b, 0, 0)),
            pl.BlockSpec((1, _LG, _D), lambda b: (0, 0, 0)),
            pl.BlockSpec((_D, _D), lambda b: (0, 0)),
            pl.BlockSpec((1, _D), lambda b: (0, 0)),
            pl.BlockSpec((2 * _D, _D), lambda b: (0, 0)),
            pl.BlockSpec((1, 2 * _D), lambda b: (0, 0)),
            pl.BlockSpec((_D, _D), lambda b: (0, 0)),
            pl.BlockSpec((1, _D), lambda b: (0, 0)),
        ],
        out_specs=[
            pl.BlockSpec((_BPS, n_out, _D), lambda b: (b, 0, 0)),
            pl.BlockSpec((_BPS, _TOPK, _D), lambda b: (b, 0, 0)),
            pl.BlockSpec((1, 1), lambda b: (0, 0), memory_space=pltpu.SMEM),
        ],
        out_shape=[
            jax.ShapeDtypeStruct((_B, n_out, _D), jnp.float32),
            jax.ShapeDtypeStruct((_B, _TOPK, _D), jnp.float32),
            jax.ShapeDtypeStruct((1, 1), jnp.float32),
        ],
        scratch_shapes=[
            pltpu.VMEM((_LG, _D), jnp.bfloat16),
            pltpu.VMEM((2 * _D, _D), jnp.bfloat16),
            pltpu.VMEM((_D, _D), jnp.bfloat16),
        ],
        interpret=interpret,
    )(x_embed, depth_feature, rows, keys, g_prompt,
      Wq, bq.reshape(1, _D), Wkv, bkv.reshape(1, 2 * _D),
      Wproj, bproj.reshape(1, _D))


def kernel(x_embed, prompt_mask, depth_feature, prompt, prompt_key,
           prompt_key_g, g_prompt, Wq, bq, Wkv, bkv, Wproj, bproj):
    idx = prompt_mask.reshape(_ROWS)
    rows, keys = _sc_gather()(prompt, prompt_key, idx)
    rows = rows.reshape(_B, _TOPK * _LEN, _D)
    keys = keys.reshape(_B, _TOPK, _D)
    prompted, bkn, sim = _dense_tc(
        x_embed, depth_feature, rows, keys, g_prompt,
        Wq, bq, Wkv, bkv, Wproj, bproj)
    return prompted, bkn, sim.reshape(())
